# baseline (speedup 1.0000x reference)
import functools
import math

import jax
import jax.numpy as jnp
from jax import lax
from jax.experimental import pallas as pl
from jax.experimental.pallas import tpu as pltpu
from jax.experimental.pallas import tpu_sc as plsc

_INFO = plsc.get_sparse_core_info()
_NC = _INFO.num_cores
_NS = _INFO.num_subcores
_NW = _NC * _NS
_L = _INFO.num_lanes

_G = 128
_GPC = 4
_CHUNK = _G * _GPC


@functools.partial(jax.jit, static_argnames=("n_chunks",))
def _run(idx2d, table, n_chunks):
    d = table.shape[1]
    b = idx2d.shape[0] * _G

    @functools.partial(
        pl.kernel,
        out_type=jax.ShapeDtypeStruct((b, d), jnp.float32),
        mesh=plsc.VectorSubcoreMesh(core_axis_name="c", subcore_axis_name="s"),
        scratch_types=[
            pltpu.VMEM((_GPC, _G), jnp.int32),
            pltpu.VMEM((_CHUNK, d), jnp.float32),
            pltpu.SemaphoreType.DMA,
        ],
        compiler_params=pltpu.CompilerParams(use_tc_tiling_on_sc=False),
    )
    def emb(idx_hbm, table_hbm, out_hbm, idx_v, rows_v, gsem):
        wid = lax.axis_index("s") * _NC + lax.axis_index("c")
        scale = jnp.float32(math.sqrt(d))

        @pl.loop(0, n_chunks)
        def _chunk(c):
            irow0 = (wid * n_chunks + c) * _GPC
            pltpu.sync_copy(idx_hbm.at[pl.ds(irow0, _GPC)], idx_v)
            descs = [
                pltpu.async_copy(
                    table_hbm.at[idx_v.at[j]],
                    rows_v.at[pl.ds(j * _G, _G)],
                    gsem,
                )
                for j in range(_GPC)
            ]
            for desc in descs:
                desc.wait()

            @pl.loop(0, _CHUNK)
            def _scale(r):
                for q in range(d // _L):
                    sl = pl.ds(q * _L, _L)
                    rows_v[r, sl] = rows_v[r, sl] * scale

            out0 = (wid * n_chunks + c) * _CHUNK
            pltpu.sync_copy(rows_v, out_hbm.at[pl.ds(out0, _CHUNK)])

    return emb(idx2d, table)


def kernel(x, table):
    batch, hist = x.shape
    d = table.shape[1]
    b = batch * hist
    assert b % (_NW * _CHUNK) == 0 and d % _L == 0
    idx2d = x.astype(jnp.int32).reshape(b // _G, _G)
    n_chunks = b // (_NW * _CHUNK)
    out = _run(idx2d, table, n_chunks)
    return out.reshape(batch, hist, d)


# speedup vs baseline: 1.1363x; 1.1363x over previous
import functools
import math

import jax
import jax.numpy as jnp
from jax import lax
from jax.experimental import pallas as pl
from jax.experimental.pallas import tpu as pltpu
from jax.experimental.pallas import tpu_sc as plsc

_INFO = plsc.get_sparse_core_info()
_NC = _INFO.num_cores
_NS = _INFO.num_subcores
_NW = _NC * _NS
_L = _INFO.num_lanes

_G = 128
_GPC = 4
_CHUNK = _G * _GPC


@functools.partial(jax.jit, static_argnames=("n_chunks",))
def _run(idx2d, table, n_chunks):
    d = table.shape[1]
    b = idx2d.shape[0] * _G
    irows_pw = n_chunks * _GPC

    @functools.partial(
        pl.kernel,
        out_type=jax.ShapeDtypeStruct((b, d), jnp.float32),
        mesh=plsc.VectorSubcoreMesh(core_axis_name="c", subcore_axis_name="s"),
        scratch_types=[
            pltpu.VMEM((irows_pw, _G), jnp.int32),
            pltpu.VMEM((_CHUNK, d), jnp.float32),
            pltpu.VMEM((_CHUNK, d), jnp.float32),
            pltpu.SemaphoreType.DMA,
            pltpu.SemaphoreType.DMA,
        ],
        compiler_params=pltpu.CompilerParams(use_tc_tiling_on_sc=False),
    )
    def emb(idx_hbm, table_hbm, out_hbm, idx_v, rows0, rows1, gsem0, gsem1):
        wid = lax.axis_index("s") * _NC + lax.axis_index("c")
        scale = jnp.float32(math.sqrt(d))
        pltpu.sync_copy(idx_hbm.at[pl.ds(wid * irows_pw, irows_pw)], idx_v)

        def fire(cc, rows, gsem):
            for j in range(_GPC):
                pltpu.async_copy(
                    table_hbm.at[idx_v.at[cc * _GPC + j]],
                    rows.at[pl.ds(j * _G, _G)],
                    gsem,
                )

        def drain(rows, gsem):
            pltpu.make_async_copy(table_hbm.at[pl.ds(0, _CHUNK)], rows, gsem).wait()

        fire(0, rows0, gsem0)

        @pl.loop(0, n_chunks, step=2)
        def _step(c):
            for bi in range(2):
                rows, gsem = (rows0, gsem0) if bi == 0 else (rows1, gsem1)
                orows, ogsem = (rows1, gsem1) if bi == 0 else (rows0, gsem0)
                cc = c + bi
                drain(rows, gsem)

                @pl.when(cc + 1 < n_chunks)
                def _prefetch():
                    fire(cc + 1, orows, ogsem)

                @pl.loop(0, _CHUNK, unroll=8)
                def _scale(r):
                    for q in range(d // _L):
                        sl = pl.ds(q * _L, _L)
                        rows[r, sl] = rows[r, sl] * scale

                pltpu.sync_copy(
                    rows, out_hbm.at[pl.ds((wid * n_chunks + cc) * _CHUNK, _CHUNK)]
                )

    return emb(idx2d, table)


def kernel(x, table):
    batch, hist = x.shape
    d = table.shape[1]
    b = batch * hist
    assert b % (_NW * _CHUNK) == 0 and d % _L == 0
    idx2d = x.astype(jnp.int32).reshape(b // _G, _G)
    n_chunks = b // (_NW * _CHUNK)
    assert n_chunks % 2 == 0
    out = _run(idx2d, table, n_chunks)
    return out.reshape(batch, hist, d)
